# Initial kernel scaffold; baseline (speedup 1.0000x reference)
import jax
import jax.numpy as jnp
from jax.experimental import pallas as pl


def kernel(hidden_states, router_weight, router_bias, gate_up_proj, gate_up_bias, down_proj, down_bias):
    raise NotImplementedError("write your pallas kernel here")


# speedup vs baseline: 3.4366x; 3.4366x over previous
import functools

import jax
import jax.numpy as jnp
from jax.experimental import pallas as pl
from jax.experimental.pallas import tpu as pltpu

NUM_EXPERTS = 8
TOP_K = 2
ALPHA = 1.702
LIMIT = 7.0

TM = 256
IBLK = 512


def _moe_body(x_ref, rwt_ref, rb_ref, wg_ref, wu_ref, bg_ref, bu_ref,
              wd_ref, db_ref, out_ref, sc_ref, xb_ref):
    e = pl.program_id(0)
    i = pl.program_id(1)
    T = x_ref.shape[0]
    nt = T // TM

    @pl.when((e == 0) & (i == 0))
    def _router_and_init():
        rwt = rwt_ref[...]
        rb = rb_ref[...]
        for t in range(nt):
            xt = x_ref[pl.ds(t * TM, TM), :]
            logits = jnp.dot(xt, rwt, preferred_element_type=jnp.float32) + rb
            lane = jax.lax.broadcasted_iota(jnp.int32, (TM, NUM_EXPERTS), 1)
            m1 = jnp.max(logits, axis=1, keepdims=True)
            idx1 = jnp.min(jnp.where(logits >= m1, lane, NUM_EXPERTS),
                           axis=1, keepdims=True)
            f1 = lane == idx1
            l2 = jnp.where(f1, -jnp.inf, logits)
            m2 = jnp.max(l2, axis=1, keepdims=True)
            idx2 = jnp.min(jnp.where(l2 >= m2, lane, NUM_EXPERTS),
                           axis=1, keepdims=True)
            f2 = lane == idx2
            p1 = 1.0 / (1.0 + jnp.exp(m2 - m1))
            p2 = 1.0 - p1
            sc_ref[pl.ds(t * TM, TM), :] = (
                jnp.where(f1, p1, 0.0) + jnp.where(f2, p2, 0.0))
            xb_ref[pl.ds(t * TM, TM), :] = xt.astype(jnp.bfloat16)
            out_ref[pl.ds(t * TM, TM), :] = jnp.zeros((TM, out_ref.shape[1]),
                                                      jnp.float32)

    wg = wg_ref[0]
    wu = wu_ref[0]
    wd = wd_ref[0]
    bg = bg_ref[0]
    bu = bu_ref[0]
    db = db_ref[0]
    for t in range(nt):
        xt = xb_ref[pl.ds(t * TM, TM), :]
        gate = jnp.dot(xt, wg, preferred_element_type=jnp.float32) + bg
        up = jnp.dot(xt, wu, preferred_element_type=jnp.float32) + bu
        gate = jnp.minimum(gate, LIMIT)
        up = jnp.clip(up, -LIMIT, LIMIT)
        glu = gate / (1.0 + jnp.exp(-ALPHA * gate))
        act = ((up + 1.0) * glu).astype(jnp.bfloat16)
        partial = jnp.dot(act, wd, preferred_element_type=jnp.float32)
        sc = sc_ref[pl.ds(t * TM, TM), :]
        lane = jax.lax.broadcasted_iota(jnp.int32, (TM, NUM_EXPERTS), 1)
        col = jnp.sum(jnp.where(lane == e, sc, 0.0), axis=1, keepdims=True)
        contrib = partial * col
        contrib = contrib + jnp.where(i == 0, 1.0, 0.0) * (col * db)
        out_ref[pl.ds(t * TM, TM), :] += contrib


@functools.partial(jax.jit, static_argnames=())
def _moe(hs, rwt, rb, wg, wu, bg, bu, wd, db):
    T, H = hs.shape
    E = NUM_EXPERTS
    I = wd.shape[1]
    ni = I // IBLK
    out, scores = pl.pallas_call(
        _moe_body,
        grid=(E, ni),
        in_specs=[
            pl.BlockSpec((T, H), lambda e, i: (0, 0)),
            pl.BlockSpec((H, E), lambda e, i: (0, 0)),
            pl.BlockSpec((1, E), lambda e, i: (0, 0)),
            pl.BlockSpec((1, H, IBLK), lambda e, i: (e, 0, i)),
            pl.BlockSpec((1, H, IBLK), lambda e, i: (e, 0, i)),
            pl.BlockSpec((1, 1, IBLK), lambda e, i: (e, 0, i)),
            pl.BlockSpec((1, 1, IBLK), lambda e, i: (e, 0, i)),
            pl.BlockSpec((1, IBLK, H), lambda e, i: (e, i, 0)),
            pl.BlockSpec((1, 1, H), lambda e, i: (e, 0, 0)),
        ],
        out_specs=[
            pl.BlockSpec((T, H), lambda e, i: (0, 0)),
            pl.BlockSpec((T, E), lambda e, i: (0, 0)),
        ],
        out_shape=[
            jax.ShapeDtypeStruct((T, H), jnp.float32),
            jax.ShapeDtypeStruct((T, E), jnp.float32),
        ],
        scratch_shapes=[pltpu.VMEM((T, H), jnp.bfloat16)],
        compiler_params=pltpu.CompilerParams(
            dimension_semantics=("arbitrary", "arbitrary"),
        ),
    )(hs, rwt, rb, wg, wu, bg, bu, wd, db)
    return out, scores


def kernel(hidden_states, router_weight, router_bias, gate_up_proj,
           gate_up_bias, down_proj, down_bias):
    B, S, H = hidden_states.shape
    E = NUM_EXPERTS
    I = down_proj.shape[1]
    hs = hidden_states.reshape(B * S, H)
    rwt = router_weight.T
    rb = router_bias.reshape(1, E)
    g4 = gate_up_proj.reshape(E, H, I, 2)
    wg = g4[..., 0].astype(jnp.bfloat16)
    wu = g4[..., 1].astype(jnp.bfloat16)
    b4 = gate_up_bias.reshape(E, 1, I, 2)
    bg = b4[..., 0]
    bu = b4[..., 1]
    wd = down_proj.astype(jnp.bfloat16)
    db = down_bias.reshape(E, 1, H)
    out, scores = _moe(hs, rwt, rb, wg, wu, bg, bu, wd, db)
    return out.reshape(B, S, H), scores


# speedup vs baseline: 3.8208x; 1.1118x over previous
import functools

import jax
import jax.numpy as jnp
from jax.experimental import pallas as pl
from jax.experimental.pallas import tpu as pltpu

NUM_EXPERTS = 8
TOP_K = 2
ALPHA = 1.702
LIMIT = 7.0

TM = 256
IBLK = 512


def _moe_body(x_ref, rwt_ref, rb_ref, wgu_ref, bgu_ref,
              wd_ref, db_ref, out_ref, sc_ref, xb_ref):
    e = pl.program_id(0)
    i = pl.program_id(1)
    T = x_ref.shape[0]
    nt = T // TM

    @pl.when((e == 0) & (i == 0))
    def _router_and_init():
        rwt = rwt_ref[...]
        rb = rb_ref[...]
        for t in range(nt):
            xt = x_ref[pl.ds(t * TM, TM), :]
            logits = jnp.dot(xt, rwt, preferred_element_type=jnp.float32) + rb
            lane = jax.lax.broadcasted_iota(jnp.int32, (TM, NUM_EXPERTS), 1)
            m1 = jnp.max(logits, axis=1, keepdims=True)
            idx1 = jnp.min(jnp.where(logits >= m1, lane, NUM_EXPERTS),
                           axis=1, keepdims=True)
            f1 = lane == idx1
            l2 = jnp.where(f1, -jnp.inf, logits)
            m2 = jnp.max(l2, axis=1, keepdims=True)
            idx2 = jnp.min(jnp.where(l2 >= m2, lane, NUM_EXPERTS),
                           axis=1, keepdims=True)
            f2 = lane == idx2
            p1 = 1.0 / (1.0 + jnp.exp(m2 - m1))
            p2 = 1.0 - p1
            sc_ref[pl.ds(t * TM, TM), :] = (
                jnp.where(f1, p1, 0.0) + jnp.where(f2, p2, 0.0))
            xb_ref[pl.ds(t * TM, TM), :] = xt.astype(jnp.bfloat16)
            out_ref[pl.ds(t * TM, TM), :] = jnp.zeros((TM, out_ref.shape[1]),
                                                      jnp.float32)

    wgu = wgu_ref[0].astype(jnp.bfloat16)
    bguT = jax.lax.transpose(bgu_ref[0], (1, 0))
    wd = wd_ref[0].astype(jnp.bfloat16)
    db = db_ref[0]
    for t in range(nt):
        xt = xb_ref[pl.ds(t * TM, TM), :]
        guT = jax.lax.dot_general(
            wgu, xt, (((0,), (1,)), ((), ())),
            preferred_element_type=jnp.float32) + bguT
        pair = pltpu.bitcast(guT.astype(jnp.bfloat16), jnp.uint32)
        gate = pltpu.bitcast(pair << 16, jnp.float32)
        up = pltpu.bitcast(pair & jnp.uint32(0xFFFF0000), jnp.float32)
        gate = jnp.minimum(gate, LIMIT)
        up = jnp.clip(up, -LIMIT, LIMIT)
        glu = gate / (1.0 + jnp.exp(-ALPHA * gate))
        act = ((up + 1.0) * glu).astype(jnp.bfloat16)
        partial = jax.lax.dot_general(
            act, wd, (((0,), (0,)), ((), ())),
            preferred_element_type=jnp.float32)
        sc = sc_ref[pl.ds(t * TM, TM), :]
        lane = jax.lax.broadcasted_iota(jnp.int32, (TM, NUM_EXPERTS), 1)
        col = jnp.sum(jnp.where(lane == e, sc, 0.0), axis=1, keepdims=True)
        contrib = partial * col
        contrib = contrib + jnp.where(i == 0, 1.0, 0.0) * (col * db)
        out_ref[pl.ds(t * TM, TM), :] += contrib


@functools.partial(jax.jit, static_argnames=())
def _moe(hs, rwt, rb, wgu, bgu, wd, db):
    T, H = hs.shape
    E = NUM_EXPERTS
    I = wd.shape[1]
    ni = I // IBLK
    out, scores = pl.pallas_call(
        _moe_body,
        grid=(E, ni),
        in_specs=[
            pl.BlockSpec((T, H), lambda e, i: (0, 0)),
            pl.BlockSpec((H, E), lambda e, i: (0, 0)),
            pl.BlockSpec((1, E), lambda e, i: (0, 0)),
            pl.BlockSpec((1, H, 2 * IBLK), lambda e, i: (e, 0, i)),
            pl.BlockSpec((1, 1, 2 * IBLK), lambda e, i: (e, 0, i)),
            pl.BlockSpec((1, IBLK, H), lambda e, i: (e, i, 0)),
            pl.BlockSpec((1, 1, H), lambda e, i: (e, 0, 0)),
        ],
        out_specs=[
            pl.BlockSpec((T, H), lambda e, i: (0, 0)),
            pl.BlockSpec((T, E), lambda e, i: (0, 0)),
        ],
        out_shape=[
            jax.ShapeDtypeStruct((T, H), jnp.float32),
            jax.ShapeDtypeStruct((T, E), jnp.float32),
        ],
        scratch_shapes=[pltpu.VMEM((T, H), jnp.bfloat16)],
        compiler_params=pltpu.CompilerParams(
            dimension_semantics=("arbitrary", "arbitrary"),
        ),
    )(hs, rwt, rb, wgu, bgu, wd, db)
    return out, scores


def kernel(hidden_states, router_weight, router_bias, gate_up_proj,
           gate_up_bias, down_proj, down_bias):
    B, S, H = hidden_states.shape
    E = NUM_EXPERTS
    I = down_proj.shape[1]
    hs = hidden_states.reshape(B * S, H)
    rwt = router_weight.T
    rb = router_bias.reshape(1, E)
    bgu = gate_up_bias.reshape(E, 1, 2 * I)
    db = down_bias.reshape(E, 1, H)
    out, scores = _moe(hs, rwt, rb, gate_up_proj, bgu, down_proj, db)
    return out.reshape(B, S, H), scores


# speedup vs baseline: 4.5903x; 1.2014x over previous
import functools

import jax
import jax.numpy as jnp
from jax.experimental import pallas as pl
from jax.experimental.pallas import tpu as pltpu

NUM_EXPERTS = 8
TOP_K = 2
ALPHA = 1.702
LIMIT = 7.0

TM = 256
TBLK = 256
ICHUNK = 1024


def _plan_body(x_ref, rwt_ref, rb_ref, sc_ref, pos_ref, ps_ref, be_ref,
               ohT_ref, cs_ref):
    T = x_ref.shape[0]
    nt = T // TM
    nc = 2 * nt
    E = NUM_EXPERTS
    NB = be_ref.shape[1]

    for t in range(nt):
        xt = x_ref[pl.ds(t * TM, TM), :]
        logits = jnp.dot(xt, rwt_ref[...],
                         preferred_element_type=jnp.float32) + rb_ref[...]
        lane = jax.lax.broadcasted_iota(jnp.int32, (TM, E), 1)
        m1 = jnp.max(logits, axis=1, keepdims=True)
        idx1 = jnp.min(jnp.where(logits >= m1, lane, E), axis=1, keepdims=True)
        f1 = lane == idx1
        l2 = jnp.where(f1, -jnp.inf, logits)
        m2 = jnp.max(l2, axis=1, keepdims=True)
        idx2 = jnp.min(jnp.where(l2 >= m2, lane, E), axis=1, keepdims=True)
        f2 = lane == idx2
        p1 = 1.0 / (1.0 + jnp.exp(m2 - m1))
        p2 = 1.0 - p1
        sc_ref[pl.ds(t * TM, TM), :] = (
            jnp.where(f1, p1, 0.0) + jnp.where(f2, p2, 0.0))
        f1T = jax.lax.transpose(f1.astype(jnp.float32), (1, 0))
        f2T = jax.lax.transpose(f2.astype(jnp.float32), (1, 0))
        ohT_ref[pl.ds(t * E, E), :] = f1T
        ohT_ref[pl.ds((nt + t) * E, E), :] = f2T
        cs_ref[pl.ds(t, 1), :] = jax.lax.transpose(
            jnp.sum(f1T, axis=1, keepdims=True), (1, 0))
        cs_ref[pl.ds(nt + t, 1), :] = jax.lax.transpose(
            jnp.sum(f2T, axis=1, keepdims=True), (1, 0))
        ps_ref[pl.ds(t, 1), :] = jax.lax.transpose(p1, (1, 0))
        ps_ref[pl.ds(nt + t, 1), :] = jax.lax.transpose(p2, (1, 0))

    cs = cs_ref[...]
    counts = jnp.sum(cs, axis=0, keepdims=True)
    pc = jnp.floor((counts + (TBLK - 1)) / TBLK) * TBLK
    re_ = jax.lax.broadcasted_iota(jnp.int32, (E, E), 0)
    ce = jax.lax.broadcasted_iota(jnp.int32, (E, E), 1)
    u_strict = (re_ < ce).astype(jnp.float32)
    gs = jnp.dot(pc, u_strict, preferred_element_type=jnp.float32)
    rc = jax.lax.broadcasted_iota(jnp.int32, (nc, nc), 0)
    cc = jax.lax.broadcasted_iota(jnp.int32, (nc, nc), 1)
    tri_nc = (rc < cc).astype(jnp.float32)
    base = jnp.dot(jax.lax.transpose(tri_nc, (1, 0)), cs,
                   preferred_element_type=jnp.float32)
    gsT = jax.lax.transpose(gs, (1, 0))
    blkpos = jax.lax.broadcasted_iota(
        jnp.int32, (1, NB), 1).astype(jnp.float32) * float(TBLK)
    be_ref[...] = (jnp.sum((gsT <= blkpos).astype(jnp.int32), axis=0,
                           keepdims=True) - 1)
    rs = jax.lax.broadcasted_iota(jnp.int32, (TM, TM), 0)
    ss = jax.lax.broadcasted_iota(jnp.int32, (TM, TM), 1)
    tri_s = (rs < ss).astype(jnp.float32)
    for c in range(nc):
        ohTc = ohT_ref[pl.ds(c * E, E), :]
        rankT = jnp.dot(ohTc, tri_s, preferred_element_type=jnp.float32)
        baseTc = jax.lax.transpose(base[c:c + 1, :], (1, 0))
        posT = jnp.sum(ohTc * (gsT + baseTc + rankT), axis=0, keepdims=True)
        pos_ref[pl.ds(c, 1), :] = posT.astype(jnp.int32)


def _plan(hs, rwt, rb, nsp):
    T, H = hs.shape
    E = NUM_EXPERTS
    nc = 2 * (T // TM)
    nb = nsp // TBLK
    return pl.pallas_call(
        _plan_body,
        grid=(1,),
        in_specs=[
            pl.BlockSpec((T, H), lambda i: (0, 0)),
            pl.BlockSpec((H, E), lambda i: (0, 0)),
            pl.BlockSpec((1, E), lambda i: (0, 0)),
        ],
        out_specs=[
            pl.BlockSpec((T, E), lambda i: (0, 0)),
            pl.BlockSpec((nc, TM), lambda i: (0, 0)),
            pl.BlockSpec((nc, TM), lambda i: (0, 0)),
            pl.BlockSpec((1, nb), lambda i: (0, 0)),
        ],
        out_shape=[
            jax.ShapeDtypeStruct((T, E), jnp.float32),
            jax.ShapeDtypeStruct((nc, TM), jnp.int32),
            jax.ShapeDtypeStruct((nc, TM), jnp.float32),
            jax.ShapeDtypeStruct((1, nb), jnp.int32),
        ],
        scratch_shapes=[
            pltpu.VMEM((nc * E, TM), jnp.float32),
            pltpu.VMEM((nc, E), jnp.float32),
        ],
    )(hs, rwt, rb)


def _experts_body(be_ref, xs_ref, wgu_ref, bgu_ref, wd_ref, db_ref, ws_ref,
                  y_ref):
    i = pl.program_id(1)
    ni = pl.num_programs(1)
    xt = xs_ref[...].astype(jnp.bfloat16)
    wgu = wgu_ref[0].astype(jnp.bfloat16)
    guT = jax.lax.dot_general(
        wgu, xt, (((0,), (1,)), ((), ())),
        preferred_element_type=jnp.float32)
    guT = guT + jax.lax.transpose(bgu_ref[0], (1, 0))
    pair = pltpu.bitcast(guT.astype(jnp.bfloat16), jnp.uint32)
    gate = pltpu.bitcast(pair << 16, jnp.float32)
    up = pltpu.bitcast(pair & jnp.uint32(0xFFFF0000), jnp.float32)
    gate = jnp.minimum(gate, LIMIT)
    up = jnp.clip(up, -LIMIT, LIMIT)
    glu = gate / (1.0 + jnp.exp(-ALPHA * gate))
    act = ((up + 1.0) * glu).astype(jnp.bfloat16)
    wd = wd_ref[0].astype(jnp.bfloat16)
    partial = jax.lax.dot_general(
        act, wd, (((0,), (0,)), ((), ())),
        preferred_element_type=jnp.float32)

    @pl.when(i == 0)
    def _init():
        y_ref[...] = partial

    @pl.when(i > 0)
    def _acc():
        y_ref[...] += partial

    @pl.when(i == ni - 1)
    def _finish():
        wcol = jax.lax.transpose(ws_ref[0], (1, 0))
        y_ref[...] = (y_ref[...] + db_ref[0]) * wcol


def _experts(be, xs, wgu, bgu, wd, db, ws3):
    nsp, H = xs.shape
    E, _, I2 = wgu.shape
    I = I2 // 2
    nb = nsp // TBLK
    ni = I // ICHUNK
    grid_spec = pltpu.PrefetchScalarGridSpec(
        num_scalar_prefetch=1,
        grid=(nb, ni),
        in_specs=[
            pl.BlockSpec((TBLK, H), lambda b, i, be: (b, 0)),
            pl.BlockSpec((1, H, 2 * ICHUNK), lambda b, i, be: (be[b], 0, i)),
            pl.BlockSpec((1, 1, 2 * ICHUNK), lambda b, i, be: (be[b], 0, i)),
            pl.BlockSpec((1, ICHUNK, H), lambda b, i, be: (be[b], i, 0)),
            pl.BlockSpec((1, 1, H), lambda b, i, be: (be[b], 0, 0)),
            pl.BlockSpec((1, 1, TBLK), lambda b, i, be: (b, 0, 0)),
        ],
        out_specs=pl.BlockSpec((TBLK, H), lambda b, i, be: (b, 0)),
    )
    return pl.pallas_call(
        _experts_body,
        grid_spec=grid_spec,
        out_shape=jax.ShapeDtypeStruct((nsp, H), jnp.float32),
        compiler_params=pltpu.CompilerParams(
            dimension_semantics=("arbitrary", "arbitrary"),
        ),
    )(be, xs, wgu, bgu, wd, db, ws3)


@functools.partial(jax.jit, static_argnames=())
def _moe(hs, rwt, rb, wgu, bgu, wd, db):
    T, H = hs.shape
    I = wd.shape[1]
    nsp = 2 * T + NUM_EXPERTS * TBLK - NUM_EXPERTS * 1
    nsp = ((nsp + TBLK - 1) // TBLK) * TBLK
    scores, pos, ps, be = _plan(hs, rwt, rb, nsp)
    pos_flat = pos.reshape(2 * T)
    tok = jnp.arange(2 * T, dtype=jnp.int32) % T
    xs = jnp.zeros((nsp, H), jnp.float32).at[pos_flat].set(hs[tok])
    ws = jnp.zeros((nsp,), jnp.float32).at[pos_flat].set(ps.reshape(2 * T))
    y = _experts(be.reshape(nsp // TBLK), xs, wgu, bgu, wd, db,
                 ws.reshape(nsp // TBLK, 1, TBLK))
    out = y[pos_flat[:T]] + y[pos_flat[T:]]
    return out, scores


def kernel(hidden_states, router_weight, router_bias, gate_up_proj,
           gate_up_bias, down_proj, down_bias):
    B, S, H = hidden_states.shape
    E = NUM_EXPERTS
    I = down_proj.shape[1]
    hs = hidden_states.reshape(B * S, H)
    rwt = router_weight.T
    rb = router_bias.reshape(1, E)
    bgu = gate_up_bias.reshape(E, 1, 2 * I)
    db = down_bias.reshape(E, 1, H)
    out, scores = _moe(hs, rwt, rb, gate_up_proj, bgu, down_proj, db)
    return out.reshape(B, S, H), scores


# speedup vs baseline: 4.6523x; 1.0135x over previous
import functools

import jax
import jax.numpy as jnp
from jax.experimental import pallas as pl
from jax.experimental.pallas import tpu as pltpu

NUM_EXPERTS = 8
TOP_K = 2
ALPHA = 1.702
LIMIT = 7.0

TM = 256
TBLK = 256
ICHUNK = 1024


def _plan_body(x_ref, rwt_ref, rb_ref, sc_ref, pos_ref, ps_ref, be_ref,
               ohT_ref, cs_ref):
    T = x_ref.shape[0]
    nt = T // TM
    nc = 2 * nt
    E = NUM_EXPERTS
    NB = be_ref.shape[1]

    for t in range(nt):
        xt = x_ref[pl.ds(t * TM, TM), :]
        logits = jnp.dot(xt, rwt_ref[...],
                         preferred_element_type=jnp.float32) + rb_ref[...]
        lane = jax.lax.broadcasted_iota(jnp.int32, (TM, E), 1)
        m1 = jnp.max(logits, axis=1, keepdims=True)
        idx1 = jnp.min(jnp.where(logits >= m1, lane, E), axis=1, keepdims=True)
        f1 = lane == idx1
        l2 = jnp.where(f1, -jnp.inf, logits)
        m2 = jnp.max(l2, axis=1, keepdims=True)
        idx2 = jnp.min(jnp.where(l2 >= m2, lane, E), axis=1, keepdims=True)
        f2 = lane == idx2
        p1 = 1.0 / (1.0 + jnp.exp(m2 - m1))
        p2 = 1.0 - p1
        sc_ref[pl.ds(t * TM, TM), :] = (
            jnp.where(f1, p1, 0.0) + jnp.where(f2, p2, 0.0))
        f1T = jax.lax.transpose(f1.astype(jnp.float32), (1, 0))
        f2T = jax.lax.transpose(f2.astype(jnp.float32), (1, 0))
        ohT_ref[pl.ds(t * E, E), :] = f1T
        ohT_ref[pl.ds((nt + t) * E, E), :] = f2T
        cs_ref[pl.ds(t, 1), :] = jax.lax.transpose(
            jnp.sum(f1T, axis=1, keepdims=True), (1, 0))
        cs_ref[pl.ds(nt + t, 1), :] = jax.lax.transpose(
            jnp.sum(f2T, axis=1, keepdims=True), (1, 0))
        ps_ref[pl.ds(t, 1), :] = jax.lax.transpose(p1, (1, 0))
        ps_ref[pl.ds(nt + t, 1), :] = jax.lax.transpose(p2, (1, 0))

    cs = cs_ref[...]
    counts = jnp.sum(cs, axis=0, keepdims=True)
    pc = jnp.floor((counts + (TBLK - 1)) / TBLK) * TBLK
    re_ = jax.lax.broadcasted_iota(jnp.int32, (E, E), 0)
    ce = jax.lax.broadcasted_iota(jnp.int32, (E, E), 1)
    u_strict = (re_ < ce).astype(jnp.float32)
    gs = jnp.dot(pc, u_strict, preferred_element_type=jnp.float32)
    rc = jax.lax.broadcasted_iota(jnp.int32, (nc, nc), 0)
    cc = jax.lax.broadcasted_iota(jnp.int32, (nc, nc), 1)
    tri_nc = (rc < cc).astype(jnp.float32)
    base = jnp.dot(jax.lax.transpose(tri_nc, (1, 0)), cs,
                   preferred_element_type=jnp.float32)
    gsT = jax.lax.transpose(gs, (1, 0))
    blkpos = jax.lax.broadcasted_iota(
        jnp.int32, (1, NB), 1).astype(jnp.float32) * float(TBLK)
    be_ref[...] = (jnp.sum((gsT <= blkpos).astype(jnp.int32), axis=0,
                           keepdims=True) - 1)
    rs = jax.lax.broadcasted_iota(jnp.int32, (TM, TM), 0)
    ss = jax.lax.broadcasted_iota(jnp.int32, (TM, TM), 1)
    tri_s = (rs < ss).astype(jnp.float32)
    for c in range(nc):
        ohTc = ohT_ref[pl.ds(c * E, E), :]
        rankT = jnp.dot(ohTc, tri_s, preferred_element_type=jnp.float32)
        baseTc = jax.lax.transpose(base[c:c + 1, :], (1, 0))
        posT = jnp.sum(ohTc * (gsT + baseTc + rankT), axis=0, keepdims=True)
        pos_ref[pl.ds(c, 1), :] = posT.astype(jnp.int32)


def _plan(hs, rwt, rb, nsp):
    T, H = hs.shape
    E = NUM_EXPERTS
    nc = 2 * (T // TM)
    nb = nsp // TBLK
    return pl.pallas_call(
        _plan_body,
        grid=(1,),
        in_specs=[
            pl.BlockSpec((T, H), lambda i: (0, 0)),
            pl.BlockSpec((H, E), lambda i: (0, 0)),
            pl.BlockSpec((1, E), lambda i: (0, 0)),
        ],
        out_specs=[
            pl.BlockSpec((T, E), lambda i: (0, 0)),
            pl.BlockSpec((nc, TM), lambda i: (0, 0)),
            pl.BlockSpec((nc, TM), lambda i: (0, 0)),
            pl.BlockSpec((1, nb), lambda i: (0, 0)),
        ],
        out_shape=[
            jax.ShapeDtypeStruct((T, E), jnp.float32),
            jax.ShapeDtypeStruct((nc, TM), jnp.int32),
            jax.ShapeDtypeStruct((nc, TM), jnp.float32),
            jax.ShapeDtypeStruct((1, nb), jnp.int32),
        ],
        scratch_shapes=[
            pltpu.VMEM((nc * E, TM), jnp.float32),
            pltpu.VMEM((nc, E), jnp.float32),
        ],
    )(hs, rwt, rb)


def _experts_body(be_ref, xs_ref, wgu_ref, bgu_ref, wd_ref, db_ref, ws_ref,
                  y_ref):
    i = pl.program_id(0)
    ni = pl.num_programs(0)
    b = pl.program_id(1)
    rows = pl.ds(b * TBLK, TBLK)
    xt = xs_ref[...].astype(jnp.bfloat16)
    wgu = wgu_ref[0].astype(jnp.bfloat16)
    guT = jax.lax.dot_general(
        wgu, xt, (((0,), (1,)), ((), ())),
        preferred_element_type=jnp.float32)
    guT = guT + jax.lax.transpose(bgu_ref[0], (1, 0))
    pair = pltpu.bitcast(guT.astype(jnp.bfloat16), jnp.uint32)
    gate = pltpu.bitcast(pair << 16, jnp.float32)
    up = pltpu.bitcast(pair & jnp.uint32(0xFFFF0000), jnp.float32)
    gate = jnp.minimum(gate, LIMIT)
    up = jnp.clip(up, -LIMIT, LIMIT)
    glu = gate / (1.0 + jnp.exp(-ALPHA * gate))
    act = ((up + 1.0) * glu).astype(jnp.bfloat16)
    wd = wd_ref[0].astype(jnp.bfloat16)
    partial = jax.lax.dot_general(
        act, wd, (((0,), (0,)), ((), ())),
        preferred_element_type=jnp.float32)

    @pl.when(i == 0)
    def _init():
        y_ref[rows, :] = partial

    @pl.when(i > 0)
    def _acc():
        y_ref[rows, :] += partial

    @pl.when(i == ni - 1)
    def _finish():
        wcol = jax.lax.transpose(ws_ref[0], (1, 0))
        y_ref[rows, :] = (y_ref[rows, :] + db_ref[0]) * wcol


def _experts(be, xs, wgu, bgu, wd, db, ws3):
    nsp, H = xs.shape
    E, _, I2 = wgu.shape
    I = I2 // 2
    nb = nsp // TBLK
    ni = I // ICHUNK
    grid_spec = pltpu.PrefetchScalarGridSpec(
        num_scalar_prefetch=1,
        grid=(ni, nb),
        in_specs=[
            pl.BlockSpec((TBLK, H), lambda i, b, be: (b, 0)),
            pl.BlockSpec((1, H, 2 * ICHUNK), lambda i, b, be: (be[b], 0, i)),
            pl.BlockSpec((1, 1, 2 * ICHUNK), lambda i, b, be: (be[b], 0, i)),
            pl.BlockSpec((1, ICHUNK, H), lambda i, b, be: (be[b], i, 0)),
            pl.BlockSpec((1, 1, H), lambda i, b, be: (be[b], 0, 0)),
            pl.BlockSpec((1, 1, TBLK), lambda i, b, be: (b, 0, 0)),
        ],
        out_specs=pl.BlockSpec((nsp, H), lambda i, b, be: (0, 0)),
    )
    return pl.pallas_call(
        _experts_body,
        grid_spec=grid_spec,
        out_shape=jax.ShapeDtypeStruct((nsp, H), jnp.float32),
        compiler_params=pltpu.CompilerParams(
            dimension_semantics=("arbitrary", "arbitrary"),
        ),
    )(be, xs, wgu, bgu, wd, db, ws3)


@functools.partial(jax.jit, static_argnames=())
def _moe(hs, rwt, rb, wgu, bgu, wd, db):
    T, H = hs.shape
    I = wd.shape[1]
    nsp = 2 * T + NUM_EXPERTS * TBLK - NUM_EXPERTS * 1
    nsp = ((nsp + TBLK - 1) // TBLK) * TBLK
    scores, pos, ps, be = _plan(hs, rwt, rb, nsp)
    pos_flat = pos.reshape(2 * T)
    tok = jnp.arange(2 * T, dtype=jnp.int32) % T
    xs = jnp.zeros((nsp, H), jnp.float32).at[pos_flat].set(hs[tok])
    ws = jnp.zeros((nsp,), jnp.float32).at[pos_flat].set(ps.reshape(2 * T))
    y = _experts(be.reshape(nsp // TBLK), xs, wgu, bgu, wd, db,
                 ws.reshape(nsp // TBLK, 1, TBLK))
    out = y[pos_flat[:T]] + y[pos_flat[T:]]
    return out, scores


def kernel(hidden_states, router_weight, router_bias, gate_up_proj,
           gate_up_bias, down_proj, down_bias):
    B, S, H = hidden_states.shape
    E = NUM_EXPERTS
    I = down_proj.shape[1]
    hs = hidden_states.reshape(B * S, H)
    rwt = router_weight.T
    rb = router_bias.reshape(1, E)
    bgu = gate_up_bias.reshape(E, 1, 2 * I)
    db = down_bias.reshape(E, 1, H)
    out, scores = _moe(hs, rwt, rb, gate_up_proj, bgu, down_proj, db)
    return out.reshape(B, S, H), scores


# speedup vs baseline: 5.5153x; 1.1855x over previous
import functools

import jax
import jax.numpy as jnp
from jax import lax
from jax.experimental import pallas as pl
from jax.experimental.pallas import tpu as pltpu
from jax.experimental.pallas import tpu_sc as plsc

NUM_EXPERTS = 8
TOP_K = 2
ALPHA = 1.702
LIMIT = 7.0

TM = 256
TBLK = 256
ICHUNK = 1024


def _plan_body(x_ref, rwt_ref, rb_ref, sc_ref, pos_ref, ps_ref, be_ref,
               ohT_ref, cs_ref):
    T = x_ref.shape[0]
    nt = T // TM
    nc = 2 * nt
    E = NUM_EXPERTS
    NB = be_ref.shape[1]

    for t in range(nt):
        xt = x_ref[pl.ds(t * TM, TM), :]
        logits = jnp.dot(xt, rwt_ref[...],
                         preferred_element_type=jnp.float32) + rb_ref[...]
        lane = jax.lax.broadcasted_iota(jnp.int32, (TM, E), 1)
        m1 = jnp.max(logits, axis=1, keepdims=True)
        idx1 = jnp.min(jnp.where(logits >= m1, lane, E), axis=1, keepdims=True)
        f1 = lane == idx1
        l2 = jnp.where(f1, -jnp.inf, logits)
        m2 = jnp.max(l2, axis=1, keepdims=True)
        idx2 = jnp.min(jnp.where(l2 >= m2, lane, E), axis=1, keepdims=True)
        f2 = lane == idx2
        p1 = 1.0 / (1.0 + jnp.exp(m2 - m1))
        p2 = 1.0 - p1
        sc_ref[pl.ds(t * TM, TM), :] = (
            jnp.where(f1, p1, 0.0) + jnp.where(f2, p2, 0.0))
        f1T = jax.lax.transpose(f1.astype(jnp.float32), (1, 0))
        f2T = jax.lax.transpose(f2.astype(jnp.float32), (1, 0))
        ohT_ref[pl.ds(t * E, E), :] = f1T
        ohT_ref[pl.ds((nt + t) * E, E), :] = f2T
        cs_ref[pl.ds(t, 1), :] = jax.lax.transpose(
            jnp.sum(f1T, axis=1, keepdims=True), (1, 0))
        cs_ref[pl.ds(nt + t, 1), :] = jax.lax.transpose(
            jnp.sum(f2T, axis=1, keepdims=True), (1, 0))
        ps_ref[pl.ds(t, 1), :] = jax.lax.transpose(p1, (1, 0))
        ps_ref[pl.ds(nt + t, 1), :] = jax.lax.transpose(p2, (1, 0))

    cs = cs_ref[...]
    counts = jnp.sum(cs, axis=0, keepdims=True)
    pc = jnp.floor((counts + (TBLK - 1)) / TBLK) * TBLK
    re_ = jax.lax.broadcasted_iota(jnp.int32, (E, E), 0)
    ce = jax.lax.broadcasted_iota(jnp.int32, (E, E), 1)
    u_strict = (re_ < ce).astype(jnp.float32)
    gs = jnp.dot(pc, u_strict, preferred_element_type=jnp.float32)
    rc = jax.lax.broadcasted_iota(jnp.int32, (nc, nc), 0)
    cc = jax.lax.broadcasted_iota(jnp.int32, (nc, nc), 1)
    tri_nc = (rc < cc).astype(jnp.float32)
    base = jnp.dot(jax.lax.transpose(tri_nc, (1, 0)), cs,
                   preferred_element_type=jnp.float32)
    gsT = jax.lax.transpose(gs, (1, 0))
    blkpos = jax.lax.broadcasted_iota(
        jnp.int32, (1, NB), 1).astype(jnp.float32) * float(TBLK)
    be_ref[...] = (jnp.sum((gsT <= blkpos).astype(jnp.int32), axis=0,
                           keepdims=True) - 1)
    rs = jax.lax.broadcasted_iota(jnp.int32, (TM, TM), 0)
    ss = jax.lax.broadcasted_iota(jnp.int32, (TM, TM), 1)
    tri_s = (rs < ss).astype(jnp.float32)
    for c in range(nc):
        ohTc = ohT_ref[pl.ds(c * E, E), :]
        rankT = jnp.dot(ohTc, tri_s, preferred_element_type=jnp.float32)
        baseTc = jax.lax.transpose(base[c:c + 1, :], (1, 0))
        posT = jnp.sum(ohTc * (gsT + baseTc + rankT), axis=0, keepdims=True)
        pos_ref[pl.ds(c, 1), :] = posT.astype(jnp.int32)


def _plan(hs, rwt, rb, nsp):
    T, H = hs.shape
    E = NUM_EXPERTS
    nc = 2 * (T // TM)
    nb = nsp // TBLK
    return pl.pallas_call(
        _plan_body,
        grid=(1,),
        in_specs=[
            pl.BlockSpec((T, H), lambda i: (0, 0)),
            pl.BlockSpec((H, E), lambda i: (0, 0)),
            pl.BlockSpec((1, E), lambda i: (0, 0)),
        ],
        out_specs=[
            pl.BlockSpec((T, E), lambda i: (0, 0)),
            pl.BlockSpec((nc, TM), lambda i: (0, 0)),
            pl.BlockSpec((nc, TM), lambda i: (0, 0)),
            pl.BlockSpec((1, nb), lambda i: (0, 0)),
        ],
        out_shape=[
            jax.ShapeDtypeStruct((T, E), jnp.float32),
            jax.ShapeDtypeStruct((nc, TM), jnp.int32),
            jax.ShapeDtypeStruct((nc, TM), jnp.float32),
            jax.ShapeDtypeStruct((1, nb), jnp.int32),
        ],
        scratch_shapes=[
            pltpu.VMEM((nc * E, TM), jnp.float32),
            pltpu.VMEM((nc, E), jnp.float32),
        ],
    )(hs, rwt, rb)


def _experts_body(be_ref, xs_ref, wgu_ref, bgu_ref, wd_ref, db_ref, ws_ref,
                  y_ref):
    i = pl.program_id(0)
    ni = pl.num_programs(0)
    b = pl.program_id(1)
    rows = pl.ds(b * TBLK, TBLK)
    xt = xs_ref[...].astype(jnp.bfloat16)
    wgu = wgu_ref[0].astype(jnp.bfloat16)
    guT = jax.lax.dot_general(
        wgu, xt, (((0,), (1,)), ((), ())),
        preferred_element_type=jnp.float32)
    guT = guT + jax.lax.transpose(bgu_ref[0], (1, 0))
    pair = pltpu.bitcast(guT.astype(jnp.bfloat16), jnp.uint32)
    gate = pltpu.bitcast(pair << 16, jnp.float32)
    up = pltpu.bitcast(pair & jnp.uint32(0xFFFF0000), jnp.float32)
    gate = jnp.minimum(gate, LIMIT)
    up = jnp.clip(up, -LIMIT, LIMIT)
    glu = gate / (1.0 + jnp.exp(-ALPHA * gate))
    act = ((up + 1.0) * glu).astype(jnp.bfloat16)
    wd = wd_ref[0].astype(jnp.bfloat16)
    partial = jax.lax.dot_general(
        act, wd, (((0,), (0,)), ((), ())),
        preferred_element_type=jnp.float32)

    @pl.when(i == 0)
    def _init():
        y_ref[rows, :] = partial

    @pl.when(i > 0)
    def _acc():
        y_ref[rows, :] += partial

    @pl.when(i == ni - 1)
    def _finish():
        wcol = jax.lax.transpose(ws_ref[0], (1, 0))
        y_ref[rows, :] = (y_ref[rows, :] + db_ref[0]) * wcol


def _experts(be, xs, wgu, bgu, wd, db, ws3):
    nsp, H = xs.shape
    E, _, I2 = wgu.shape
    I = I2 // 2
    nb = nsp // TBLK
    ni = I // ICHUNK
    grid_spec = pltpu.PrefetchScalarGridSpec(
        num_scalar_prefetch=1,
        grid=(ni, nb),
        in_specs=[
            pl.BlockSpec((TBLK, H), lambda i, b, be: (b, 0)),
            pl.BlockSpec((1, H, 2 * ICHUNK), lambda i, b, be: (be[b], 0, i)),
            pl.BlockSpec((1, 1, 2 * ICHUNK), lambda i, b, be: (be[b], 0, i)),
            pl.BlockSpec((1, ICHUNK, H), lambda i, b, be: (be[b], i, 0)),
            pl.BlockSpec((1, 1, H), lambda i, b, be: (be[b], 0, 0)),
            pl.BlockSpec((1, 1, TBLK), lambda i, b, be: (b, 0, 0)),
        ],
        out_specs=pl.BlockSpec((nsp, H), lambda i, b, be: (0, 0)),
    )
    return pl.pallas_call(
        _experts_body,
        grid_spec=grid_spec,
        out_shape=jax.ShapeDtypeStruct((nsp, H), jnp.float32),
        compiler_params=pltpu.CompilerParams(
            dimension_semantics=("arbitrary", "arbitrary"),
        ),
    )(be, xs, wgu, bgu, wd, db, ws3)


def _dispatch(pos_flat, ps_flat, hs, nsp):
    T, H = hs.shape
    S2 = pos_flat.shape[0]
    per_w = S2 // 32
    mesh = plsc.VectorSubcoreMesh(core_axis_name="c", subcore_axis_name="s")

    @functools.partial(
        pl.kernel, mesh=mesh,
        out_type=[
            jax.ShapeDtypeStruct((nsp, H), jnp.float32),
            jax.ShapeDtypeStruct((nsp,), jnp.float32),
        ],
        scratch_types=[
            pltpu.VMEM((per_w,), jnp.int32),
            pltpu.VMEM((per_w, H), jnp.float32),
            pltpu.VMEM((per_w,), jnp.float32),
            pltpu.SemaphoreType.DMA,
        ],
    )
    def k(pos_hbm, ps_hbm, x_hbm, xs_hbm, ws_hbm, idx_v, rows_v, p_v, sem):
        wid = lax.axis_index("s") * 2 + lax.axis_index("c")
        s0 = wid * per_w
        t0 = lax.rem(s0, T)
        pltpu.sync_copy(pos_hbm.at[pl.ds(s0, per_w)], idx_v)
        pltpu.sync_copy(x_hbm.at[pl.ds(t0, per_w)], rows_v)
        pltpu.async_copy(rows_v, xs_hbm.at[idx_v], sem).wait()
        pltpu.sync_copy(ps_hbm.at[pl.ds(s0, per_w)], p_v)
        pltpu.async_copy(p_v, ws_hbm.at[idx_v], sem).wait()

    return k(pos_flat, ps_flat, hs)


def _combine(pos_flat, y, T):
    nsp, H = y.shape
    per_w = T // 32
    mesh = plsc.VectorSubcoreMesh(core_axis_name="c", subcore_axis_name="s")

    @functools.partial(
        pl.kernel, mesh=mesh,
        out_type=jax.ShapeDtypeStruct((T, H), jnp.float32),
        scratch_types=[
            pltpu.VMEM((per_w,), jnp.int32),
            pltpu.VMEM((per_w,), jnp.int32),
            pltpu.VMEM((per_w, H), jnp.float32),
            pltpu.VMEM((per_w, H), jnp.float32),
            pltpu.SemaphoreType.DMA,
        ],
    )
    def k(pos_hbm, y_hbm, out_hbm, i1_v, i2_v, r1_v, r2_v, sem):
        wid = lax.axis_index("s") * 2 + lax.axis_index("c")
        t0 = wid * per_w
        pltpu.sync_copy(pos_hbm.at[pl.ds(t0, per_w)], i1_v)
        pltpu.sync_copy(pos_hbm.at[pl.ds(T + t0, per_w)], i2_v)
        pltpu.async_copy(y_hbm.at[i1_v], r1_v, sem).wait()
        pltpu.async_copy(y_hbm.at[i2_v], r2_v, sem).wait()

        def row(i, _):
            for j in range(H // 16):
                cols = pl.ds(j * 16, 16)
                r1_v[i, cols] = r1_v[i, cols] + r2_v[i, cols]
            return 0

        lax.fori_loop(0, per_w, row, 0)
        pltpu.sync_copy(r1_v, out_hbm.at[pl.ds(t0, per_w)])

    return k(pos_flat, y)


@functools.partial(jax.jit, static_argnames=())
def _moe(hs, rwt, rb, wgu, bgu, wd, db):
    T, H = hs.shape
    I = wd.shape[1]
    nsp = 2 * T + NUM_EXPERTS * TBLK - NUM_EXPERTS * 1
    nsp = ((nsp + TBLK - 1) // TBLK) * TBLK
    scores, pos, ps, be = _plan(hs, rwt, rb, nsp)
    pos_flat = pos.reshape(2 * T)
    xs, ws = _dispatch(pos_flat, ps.reshape(2 * T), hs, nsp)
    y = _experts(be.reshape(nsp // TBLK), xs, wgu, bgu, wd, db,
                 ws.reshape(nsp // TBLK, 1, TBLK))
    out = _combine(pos_flat, y, T)
    return out, scores


def kernel(hidden_states, router_weight, router_bias, gate_up_proj,
           gate_up_bias, down_proj, down_bias):
    B, S, H = hidden_states.shape
    E = NUM_EXPERTS
    I = down_proj.shape[1]
    hs = hidden_states.reshape(B * S, H)
    rwt = router_weight.T
    rb = router_bias.reshape(1, E)
    bgu = gate_up_bias.reshape(E, 1, 2 * I)
    db = down_bias.reshape(E, 1, H)
    out, scores = _moe(hs, rwt, rb, gate_up_proj, bgu, down_proj, db)
    return out.reshape(B, S, H), scores


# speedup vs baseline: 6.1858x; 1.1216x over previous
import functools

import jax
import jax.numpy as jnp
from jax import lax
from jax.experimental import pallas as pl
from jax.experimental.pallas import tpu as pltpu
from jax.experimental.pallas import tpu_sc as plsc

NUM_EXPERTS = 8
TOP_K = 2
ALPHA = 1.702
LIMIT = 7.0

TM = 256
TBLK = 256
ICHUNK = 1024


def _plan_body(x_ref, rwt_ref, rb_ref, sc_ref, pos_ref, ps_ref, be_ref,
               ohT_ref, cs_ref):
    T = x_ref.shape[0]
    nt = T // TM
    nc = 2 * nt
    E = NUM_EXPERTS
    NB = be_ref.shape[1]

    for t in range(nt):
        xt = x_ref[pl.ds(t * TM, TM), :]
        logits = jnp.dot(xt, rwt_ref[...],
                         preferred_element_type=jnp.float32) + rb_ref[...]
        lane = jax.lax.broadcasted_iota(jnp.int32, (TM, E), 1)
        m1 = jnp.max(logits, axis=1, keepdims=True)
        idx1 = jnp.min(jnp.where(logits >= m1, lane, E), axis=1, keepdims=True)
        f1 = lane == idx1
        l2 = jnp.where(f1, -jnp.inf, logits)
        m2 = jnp.max(l2, axis=1, keepdims=True)
        idx2 = jnp.min(jnp.where(l2 >= m2, lane, E), axis=1, keepdims=True)
        f2 = lane == idx2
        p1 = 1.0 / (1.0 + jnp.exp(m2 - m1))
        p2 = 1.0 - p1
        sc_ref[pl.ds(t * TM, TM), :] = (
            jnp.where(f1, p1, 0.0) + jnp.where(f2, p2, 0.0))
        f1T = jax.lax.transpose(f1.astype(jnp.float32), (1, 0))
        f2T = jax.lax.transpose(f2.astype(jnp.float32), (1, 0))
        ohT_ref[pl.ds(t * E, E), :] = f1T
        ohT_ref[pl.ds((nt + t) * E, E), :] = f2T
        cs_ref[pl.ds(t, 1), :] = jax.lax.transpose(
            jnp.sum(f1T, axis=1, keepdims=True), (1, 0))
        cs_ref[pl.ds(nt + t, 1), :] = jax.lax.transpose(
            jnp.sum(f2T, axis=1, keepdims=True), (1, 0))
        ps_ref[pl.ds(t, 1), :] = jax.lax.transpose(p1, (1, 0))
        ps_ref[pl.ds(nt + t, 1), :] = jax.lax.transpose(p2, (1, 0))

    cs = cs_ref[...]
    counts = jnp.sum(cs, axis=0, keepdims=True)
    pc = jnp.floor((counts + (TBLK - 1)) / TBLK) * TBLK
    re_ = jax.lax.broadcasted_iota(jnp.int32, (E, E), 0)
    ce = jax.lax.broadcasted_iota(jnp.int32, (E, E), 1)
    u_strict = (re_ < ce).astype(jnp.float32)
    gs = jnp.dot(pc, u_strict, preferred_element_type=jnp.float32)
    rc = jax.lax.broadcasted_iota(jnp.int32, (nc, nc), 0)
    cc = jax.lax.broadcasted_iota(jnp.int32, (nc, nc), 1)
    tri_nc = (rc < cc).astype(jnp.float32)
    base = jnp.dot(jax.lax.transpose(tri_nc, (1, 0)), cs,
                   preferred_element_type=jnp.float32)
    gsT = jax.lax.transpose(gs, (1, 0))
    blkpos = jax.lax.broadcasted_iota(
        jnp.int32, (1, NB), 1).astype(jnp.float32) * float(TBLK)
    be_ref[...] = (jnp.sum((gsT <= blkpos).astype(jnp.int32), axis=0,
                           keepdims=True) - 1)
    rs = jax.lax.broadcasted_iota(jnp.int32, (TM, TM), 0)
    ss = jax.lax.broadcasted_iota(jnp.int32, (TM, TM), 1)
    tri_s = (rs < ss).astype(jnp.float32)
    for c in range(nc):
        ohTc = ohT_ref[pl.ds(c * E, E), :]
        rankT = jnp.dot(ohTc, tri_s, preferred_element_type=jnp.float32)
        baseTc = jax.lax.transpose(base[c:c + 1, :], (1, 0))
        posT = jnp.sum(ohTc * (gsT + baseTc + rankT), axis=0, keepdims=True)
        pos_ref[pl.ds(c, 1), :] = posT.astype(jnp.int32)


def _plan(hs, rwt, rb, nsp):
    T, H = hs.shape
    E = NUM_EXPERTS
    nc = 2 * (T // TM)
    nb = nsp // TBLK
    return pl.pallas_call(
        _plan_body,
        grid=(1,),
        in_specs=[
            pl.BlockSpec((T, H), lambda i: (0, 0)),
            pl.BlockSpec((H, E), lambda i: (0, 0)),
            pl.BlockSpec((1, E), lambda i: (0, 0)),
        ],
        out_specs=[
            pl.BlockSpec((T, E), lambda i: (0, 0)),
            pl.BlockSpec((nc, TM), lambda i: (0, 0)),
            pl.BlockSpec((nc, TM), lambda i: (0, 0)),
            pl.BlockSpec((1, nb), lambda i: (0, 0)),
        ],
        out_shape=[
            jax.ShapeDtypeStruct((T, E), jnp.float32),
            jax.ShapeDtypeStruct((nc, TM), jnp.int32),
            jax.ShapeDtypeStruct((nc, TM), jnp.float32),
            jax.ShapeDtypeStruct((1, nb), jnp.int32),
        ],
        scratch_shapes=[
            pltpu.VMEM((nc * E, TM), jnp.float32),
            pltpu.VMEM((nc, E), jnp.float32),
        ],
    )(hs, rwt, rb)


def _experts_body(be_ref, xs_ref, wgu_ref, bgu_ref, wd_ref, db_ref, ws_ref,
                  y_ref):
    xt = xs_ref[...].astype(jnp.bfloat16)
    wgu = wgu_ref[0].astype(jnp.bfloat16)
    guT = jax.lax.dot_general(
        wgu, xt, (((0,), (1,)), ((), ())),
        preferred_element_type=jnp.float32)
    guT = guT + jax.lax.transpose(bgu_ref[0], (1, 0))
    pair = pltpu.bitcast(guT.astype(jnp.bfloat16), jnp.uint32)
    gate = pltpu.bitcast(pair << 16, jnp.float32)
    up = pltpu.bitcast(pair & jnp.uint32(0xFFFF0000), jnp.float32)
    gate = jnp.minimum(gate, LIMIT)
    up = jnp.clip(up, -LIMIT, LIMIT)
    glu = gate / (1.0 + jnp.exp(-ALPHA * gate))
    act = ((up + 1.0) * glu).astype(jnp.bfloat16)
    wd = wd_ref[0].astype(jnp.bfloat16)
    partial = jax.lax.dot_general(
        act, wd, (((0,), (0,)), ((), ())),
        preferred_element_type=jnp.float32)
    wcol = jax.lax.transpose(ws_ref[0], (1, 0))
    y_ref[...] = (partial + db_ref[0]) * wcol


def _experts(be, xs, wgu, bgu, wd, db, ws3):
    nsp, H = xs.shape
    E, _, I2 = wgu.shape
    I = I2 // 2
    nb = nsp // TBLK
    grid_spec = pltpu.PrefetchScalarGridSpec(
        num_scalar_prefetch=1,
        grid=(nb,),
        in_specs=[
            pl.BlockSpec((TBLK, H), lambda b, be: (b, 0)),
            pl.BlockSpec((1, H, 2 * I), lambda b, be: (be[b], 0, 0)),
            pl.BlockSpec((1, 1, 2 * I), lambda b, be: (be[b], 0, 0)),
            pl.BlockSpec((1, I, H), lambda b, be: (be[b], 0, 0)),
            pl.BlockSpec((1, 1, H), lambda b, be: (be[b], 0, 0)),
            pl.BlockSpec((1, 1, TBLK), lambda b, be: (b, 0, 0)),
        ],
        out_specs=pl.BlockSpec((TBLK, H), lambda b, be: (b, 0)),
    )
    return pl.pallas_call(
        _experts_body,
        grid_spec=grid_spec,
        out_shape=jax.ShapeDtypeStruct((nsp, H), jnp.float32),
        compiler_params=pltpu.CompilerParams(
            dimension_semantics=("arbitrary",),
        ),
    )(be, xs, wgu, bgu, wd, db, ws3)


def _dispatch(pos_flat, ps_flat, hs, nsp):
    T, H = hs.shape
    S2 = pos_flat.shape[0]
    per_w = S2 // 32
    mesh = plsc.VectorSubcoreMesh(core_axis_name="c", subcore_axis_name="s")

    @functools.partial(
        pl.kernel, mesh=mesh,
        out_type=[
            jax.ShapeDtypeStruct((nsp, H), jnp.float32),
            jax.ShapeDtypeStruct((nsp,), jnp.float32),
        ],
        scratch_types=[
            pltpu.VMEM((per_w,), jnp.int32),
            pltpu.VMEM((per_w, H), jnp.float32),
            pltpu.VMEM((per_w,), jnp.float32),
            pltpu.SemaphoreType.DMA,
        ],
    )
    def k(pos_hbm, ps_hbm, x_hbm, xs_hbm, ws_hbm, idx_v, rows_v, p_v, sem):
        wid = lax.axis_index("s") * 2 + lax.axis_index("c")
        s0 = wid * per_w
        t0 = lax.rem(s0, T)
        pltpu.sync_copy(pos_hbm.at[pl.ds(s0, per_w)], idx_v)
        pltpu.sync_copy(x_hbm.at[pl.ds(t0, per_w)], rows_v)
        pltpu.async_copy(rows_v, xs_hbm.at[idx_v], sem).wait()
        pltpu.sync_copy(ps_hbm.at[pl.ds(s0, per_w)], p_v)
        pltpu.async_copy(p_v, ws_hbm.at[idx_v], sem).wait()

    return k(pos_flat, ps_flat, hs)


def _combine(pos_flat, y, T):
    nsp, H = y.shape
    per_w = T // 32
    mesh = plsc.VectorSubcoreMesh(core_axis_name="c", subcore_axis_name="s")

    @functools.partial(
        pl.kernel, mesh=mesh,
        out_type=jax.ShapeDtypeStruct((T, H), jnp.float32),
        scratch_types=[
            pltpu.VMEM((per_w,), jnp.int32),
            pltpu.VMEM((per_w,), jnp.int32),
            pltpu.VMEM((per_w, H), jnp.float32),
            pltpu.VMEM((per_w, H), jnp.float32),
            pltpu.SemaphoreType.DMA,
        ],
    )
    def k(pos_hbm, y_hbm, out_hbm, i1_v, i2_v, r1_v, r2_v, sem):
        wid = lax.axis_index("s") * 2 + lax.axis_index("c")
        t0 = wid * per_w
        pltpu.sync_copy(pos_hbm.at[pl.ds(t0, per_w)], i1_v)
        pltpu.sync_copy(pos_hbm.at[pl.ds(T + t0, per_w)], i2_v)
        pltpu.async_copy(y_hbm.at[i1_v], r1_v, sem).wait()
        pltpu.async_copy(y_hbm.at[i2_v], r2_v, sem).wait()

        def row(i, _):
            for j in range(H // 16):
                cols = pl.ds(j * 16, 16)
                r1_v[i, cols] = r1_v[i, cols] + r2_v[i, cols]
            return 0

        lax.fori_loop(0, per_w, row, 0)
        pltpu.sync_copy(r1_v, out_hbm.at[pl.ds(t0, per_w)])

    return k(pos_flat, y)


@functools.partial(jax.jit, static_argnames=())
def _moe(hs, rwt, rb, wgu, bgu, wd, db):
    T, H = hs.shape
    I = wd.shape[1]
    nsp = 2 * T + NUM_EXPERTS * TBLK - NUM_EXPERTS * 1
    nsp = ((nsp + TBLK - 1) // TBLK) * TBLK
    scores, pos, ps, be = _plan(hs, rwt, rb, nsp)
    pos_flat = pos.reshape(2 * T)
    xs, ws = _dispatch(pos_flat, ps.reshape(2 * T), hs, nsp)
    y = _experts(be.reshape(nsp // TBLK), xs, wgu, bgu, wd, db,
                 ws.reshape(nsp // TBLK, 1, TBLK))
    out = _combine(pos_flat, y, T)
    return out, scores


def kernel(hidden_states, router_weight, router_bias, gate_up_proj,
           gate_up_bias, down_proj, down_bias):
    B, S, H = hidden_states.shape
    E = NUM_EXPERTS
    I = down_proj.shape[1]
    hs = hidden_states.reshape(B * S, H)
    rwt = router_weight.T
    rb = router_bias.reshape(1, E)
    bgu = gate_up_bias.reshape(E, 1, 2 * I)
    db = down_bias.reshape(E, 1, H)
    out, scores = _moe(hs, rwt, rb, gate_up_proj, bgu, down_proj, db)
    return out.reshape(B, S, H), scores


# speedup vs baseline: 6.2645x; 1.0127x over previous
import functools

import jax
import jax.numpy as jnp
from jax import lax
from jax.experimental import pallas as pl
from jax.experimental.pallas import tpu as pltpu
from jax.experimental.pallas import tpu_sc as plsc

NUM_EXPERTS = 8
TOP_K = 2
ALPHA = 1.702
LIMIT = 7.0

TM = 256
TBLK = 256
ICHUNK = 1024


def _plan_body(x_ref, rwt_ref, rb_ref, sc_ref, pos_ref, ps_ref, be_ref,
               ohT_ref, cs_ref):
    T = x_ref.shape[0]
    nt = T // TM
    nc = 2 * nt
    E = NUM_EXPERTS
    NB = be_ref.shape[1]

    for t in range(nt):
        xt = x_ref[pl.ds(t * TM, TM), :]
        logits = jnp.dot(xt, rwt_ref[...],
                         preferred_element_type=jnp.float32) + rb_ref[...]
        lane = jax.lax.broadcasted_iota(jnp.int32, (TM, E), 1)
        m1 = jnp.max(logits, axis=1, keepdims=True)
        idx1 = jnp.min(jnp.where(logits >= m1, lane, E), axis=1, keepdims=True)
        f1 = lane == idx1
        l2 = jnp.where(f1, -jnp.inf, logits)
        m2 = jnp.max(l2, axis=1, keepdims=True)
        idx2 = jnp.min(jnp.where(l2 >= m2, lane, E), axis=1, keepdims=True)
        f2 = lane == idx2
        p1 = 1.0 / (1.0 + jnp.exp(m2 - m1))
        p2 = 1.0 - p1
        sc_ref[pl.ds(t * TM, TM), :] = (
            jnp.where(f1, p1, 0.0) + jnp.where(f2, p2, 0.0))
        f1T = jax.lax.transpose(f1.astype(jnp.float32), (1, 0))
        f2T = jax.lax.transpose(f2.astype(jnp.float32), (1, 0))
        ohT_ref[pl.ds(t * E, E), :] = f1T
        ohT_ref[pl.ds((nt + t) * E, E), :] = f2T
        cs_ref[pl.ds(t, 1), :] = jax.lax.transpose(
            jnp.sum(f1T, axis=1, keepdims=True), (1, 0))
        cs_ref[pl.ds(nt + t, 1), :] = jax.lax.transpose(
            jnp.sum(f2T, axis=1, keepdims=True), (1, 0))
        ps_ref[pl.ds(t, 1), :] = jax.lax.transpose(p1, (1, 0))
        ps_ref[pl.ds(nt + t, 1), :] = jax.lax.transpose(p2, (1, 0))

    cs = cs_ref[...]
    counts = jnp.sum(cs, axis=0, keepdims=True)
    pc = jnp.floor((counts + (TBLK - 1)) / TBLK) * TBLK
    re_ = jax.lax.broadcasted_iota(jnp.int32, (E, E), 0)
    ce = jax.lax.broadcasted_iota(jnp.int32, (E, E), 1)
    u_strict = (re_ < ce).astype(jnp.float32)
    gs = jnp.dot(pc, u_strict, preferred_element_type=jnp.float32)
    rc = jax.lax.broadcasted_iota(jnp.int32, (nc, nc), 0)
    cc = jax.lax.broadcasted_iota(jnp.int32, (nc, nc), 1)
    tri_nc = (rc < cc).astype(jnp.float32)
    base = jnp.dot(jax.lax.transpose(tri_nc, (1, 0)), cs,
                   preferred_element_type=jnp.float32)
    gsT = jax.lax.transpose(gs, (1, 0))
    blkpos = jax.lax.broadcasted_iota(
        jnp.int32, (1, NB), 1).astype(jnp.float32) * float(TBLK)
    be_ref[...] = (jnp.sum((gsT <= blkpos).astype(jnp.int32), axis=0,
                           keepdims=True) - 1)
    rs = jax.lax.broadcasted_iota(jnp.int32, (TM, TM), 0)
    ss = jax.lax.broadcasted_iota(jnp.int32, (TM, TM), 1)
    tri_s = (rs < ss).astype(jnp.float32)
    for c in range(nc):
        ohTc = ohT_ref[pl.ds(c * E, E), :]
        rankT = jnp.dot(ohTc, tri_s, preferred_element_type=jnp.float32)
        baseTc = jax.lax.transpose(base[c:c + 1, :], (1, 0))
        posT = jnp.sum(ohTc * (gsT + baseTc + rankT), axis=0, keepdims=True)
        pos_ref[pl.ds(c, 1), :] = posT.astype(jnp.int32)


def _plan(hs, rwt, rb, nsp):
    T, H = hs.shape
    E = NUM_EXPERTS
    nc = 2 * (T // TM)
    nb = nsp // TBLK
    return pl.pallas_call(
        _plan_body,
        grid=(1,),
        in_specs=[
            pl.BlockSpec((T, H), lambda i: (0, 0)),
            pl.BlockSpec((H, E), lambda i: (0, 0)),
            pl.BlockSpec((1, E), lambda i: (0, 0)),
        ],
        out_specs=[
            pl.BlockSpec((T, E), lambda i: (0, 0)),
            pl.BlockSpec((nc, TM), lambda i: (0, 0)),
            pl.BlockSpec((nc, TM), lambda i: (0, 0)),
            pl.BlockSpec((1, nb), lambda i: (0, 0)),
        ],
        out_shape=[
            jax.ShapeDtypeStruct((T, E), jnp.float32),
            jax.ShapeDtypeStruct((nc, TM), jnp.int32),
            jax.ShapeDtypeStruct((nc, TM), jnp.float32),
            jax.ShapeDtypeStruct((1, nb), jnp.int32),
        ],
        scratch_shapes=[
            pltpu.VMEM((nc * E, TM), jnp.float32),
            pltpu.VMEM((nc, E), jnp.float32),
        ],
    )(hs, rwt, rb)


def _experts_body(be_ref, xs_ref, wgu_ref, bgu_ref, wd_ref, db_ref, ws_ref,
                  y_ref):
    xt = xs_ref[...].astype(jnp.bfloat16)
    wgu = wgu_ref[0].astype(jnp.bfloat16)
    guT = jax.lax.dot_general(
        wgu, xt, (((0,), (1,)), ((), ())),
        preferred_element_type=jnp.float32)
    guT = guT + jax.lax.transpose(bgu_ref[0], (1, 0))
    pair = pltpu.bitcast(guT.astype(jnp.bfloat16), jnp.uint32)
    gate = pltpu.bitcast(pair << 16, jnp.float32)
    up = pltpu.bitcast(pair & jnp.uint32(0xFFFF0000), jnp.float32)
    gate = jnp.minimum(gate, LIMIT)
    up = jnp.clip(up, -LIMIT, LIMIT)
    glu = gate / (1.0 + jnp.exp(-ALPHA * gate))
    act = ((up + 1.0) * glu).astype(jnp.bfloat16)
    wd = wd_ref[0].astype(jnp.bfloat16)
    partial = jax.lax.dot_general(
        act, wd, (((0,), (0,)), ((), ())),
        preferred_element_type=jnp.float32)
    wcol = jax.lax.transpose(ws_ref[0], (1, 0))
    y_ref[...] = (partial + db_ref[0]) * wcol


def _experts(be, xs, wgu, bgu, wd, db, ws3):
    nsp, H = xs.shape
    E, _, I2 = wgu.shape
    I = I2 // 2
    nb = nsp // TBLK
    grid_spec = pltpu.PrefetchScalarGridSpec(
        num_scalar_prefetch=1,
        grid=(nb,),
        in_specs=[
            pl.BlockSpec((TBLK, H), lambda b, be: (b, 0)),
            pl.BlockSpec((1, H, 2 * I), lambda b, be: (be[b], 0, 0)),
            pl.BlockSpec((1, 1, 2 * I), lambda b, be: (be[b], 0, 0)),
            pl.BlockSpec((1, I, H), lambda b, be: (be[b], 0, 0)),
            pl.BlockSpec((1, 1, H), lambda b, be: (be[b], 0, 0)),
            pl.BlockSpec((1, 1, TBLK), lambda b, be: (b, 0, 0)),
        ],
        out_specs=pl.BlockSpec((TBLK, H), lambda b, be: (b, 0)),
    )
    return pl.pallas_call(
        _experts_body,
        grid_spec=grid_spec,
        out_shape=jax.ShapeDtypeStruct((nsp, H), jnp.float32),
        compiler_params=pltpu.CompilerParams(
            dimension_semantics=("arbitrary",),
        ),
    )(be, xs, wgu, bgu, wd, db, ws3)


def _dispatch(pos2, ps2, hs, nsp):
    T, H = hs.shape
    nc, TMc = pos2.shape
    per_w = nc * TMc // 32
    half = per_w // 2
    mesh = plsc.VectorSubcoreMesh(core_axis_name="c", subcore_axis_name="s")

    @functools.partial(
        pl.kernel, mesh=mesh,
        out_type=[
            jax.ShapeDtypeStruct((nsp, H), jnp.float32),
            jax.ShapeDtypeStruct((nsp,), jnp.float32),
        ],
        scratch_types=[
            pltpu.VMEM((per_w,), jnp.int32),
            pltpu.VMEM((half,), jnp.int32),
            pltpu.VMEM((half,), jnp.int32),
            pltpu.VMEM((half, H), jnp.float32),
            pltpu.VMEM((half, H), jnp.float32),
            pltpu.VMEM((per_w,), jnp.float32),
            pltpu.SemaphoreType.DMA,
            pltpu.SemaphoreType.DMA,
            pltpu.SemaphoreType.DMA,
        ],
    )
    def k(pos_hbm, ps_hbm, x_hbm, xs_hbm, ws_hbm,
          idx_v, idx_a, idx_b, rows_a, rows_b, p_v, sem_a, sem_b, sem_s):
        wid = lax.axis_index("s") * 2 + lax.axis_index("c")
        s0 = wid * per_w
        row = wid // 2
        col = lax.rem(wid, 2) * per_w
        t0 = lax.rem(s0, T)
        pltpu.sync_copy(pos_hbm.at[row, pl.ds(col, per_w)], idx_v)
        pltpu.sync_copy(pos_hbm.at[row, pl.ds(col, half)], idx_a)
        pltpu.sync_copy(pos_hbm.at[row, pl.ds(col + half, half)], idx_b)
        pltpu.sync_copy(ps_hbm.at[row, pl.ds(col, per_w)], p_v)
        cp_a = pltpu.async_copy(x_hbm.at[pl.ds(t0, half)], rows_a, sem_a)
        cp_b = pltpu.async_copy(x_hbm.at[pl.ds(t0 + half, half)], rows_b,
                                sem_b)
        pltpu.async_copy(p_v, ws_hbm.at[idx_v], sem_s).wait()
        cp_a.wait()
        sc_a = pltpu.async_copy(rows_a, xs_hbm.at[idx_a], sem_a)
        cp_b.wait()
        pltpu.async_copy(rows_b, xs_hbm.at[idx_b], sem_b).wait()
        sc_a.wait()

    return k(pos2, ps2, hs)


def _combine(pos2, y, T):
    nsp, H = y.shape
    nc, TMc = pos2.shape
    per_w = T // 32
    rows_per = TMc // per_w
    mesh = plsc.VectorSubcoreMesh(core_axis_name="c", subcore_axis_name="s")

    @functools.partial(
        pl.kernel, mesh=mesh,
        out_type=jax.ShapeDtypeStruct((T, H), jnp.float32),
        scratch_types=[
            pltpu.VMEM((per_w,), jnp.int32),
            pltpu.VMEM((per_w,), jnp.int32),
            pltpu.VMEM((per_w, H), jnp.float32),
            pltpu.VMEM((per_w, H), jnp.float32),
            pltpu.SemaphoreType.DMA,
        ],
    )
    def k(pos_hbm, y_hbm, out_hbm, i1_v, i2_v, r1_v, r2_v, sem):
        wid = lax.axis_index("s") * 2 + lax.axis_index("c")
        t0 = wid * per_w
        row = wid // rows_per
        col = lax.rem(wid, rows_per) * per_w
        pltpu.sync_copy(pos_hbm.at[row, pl.ds(col, per_w)], i1_v)
        pltpu.sync_copy(pos_hbm.at[nc // 2 + row, pl.ds(col, per_w)], i2_v)
        pltpu.async_copy(y_hbm.at[i1_v], r1_v, sem).wait()
        pltpu.async_copy(y_hbm.at[i2_v], r2_v, sem).wait()

        def row(i, _):
            for j in range(H // 16):
                cols = pl.ds(j * 16, 16)
                r1_v[i, cols] = r1_v[i, cols] + r2_v[i, cols]
            return 0

        lax.fori_loop(0, per_w, row, 0)
        pltpu.sync_copy(r1_v, out_hbm.at[pl.ds(t0, per_w)])

    return k(pos2, y)


@functools.partial(jax.jit, static_argnames=())
def _moe(hs, rwt, rb, wgu, bgu, wd, db):
    T, H = hs.shape
    I = wd.shape[1]
    nsp = 2 * T + NUM_EXPERTS * TBLK - NUM_EXPERTS * 1
    nsp = ((nsp + TBLK - 1) // TBLK) * TBLK
    scores, pos, ps, be = _plan(hs, rwt, rb, nsp)
    xs, ws = _dispatch(pos, ps, hs, nsp)
    y = _experts(be.reshape(nsp // TBLK), xs, wgu, bgu, wd, db,
                 ws.reshape(nsp // TBLK, 1, TBLK))
    out = _combine(pos, y, T)
    return out, scores


def kernel(hidden_states, router_weight, router_bias, gate_up_proj,
           gate_up_bias, down_proj, down_bias):
    B, S, H = hidden_states.shape
    E = NUM_EXPERTS
    I = down_proj.shape[1]
    hs = hidden_states.reshape(B * S, H)
    rwt = router_weight.T
    rb = router_bias.reshape(1, E)
    bgu = gate_up_bias.reshape(E, 1, 2 * I)
    db = down_bias.reshape(E, 1, H)
    out, scores = _moe(hs, rwt, rb, gate_up_proj, bgu, down_proj, db)
    return out.reshape(B, S, H), scores


# speedup vs baseline: 6.2879x; 1.0037x over previous
import functools

import jax
import jax.numpy as jnp
from jax import lax
from jax.experimental import pallas as pl
from jax.experimental.pallas import tpu as pltpu
from jax.experimental.pallas import tpu_sc as plsc

NUM_EXPERTS = 8
TOP_K = 2
ALPHA = 1.702
LIMIT = 7.0

TM = 256
TBLK = 256


def _plan_body(x_ref, rwt_ref, rb_ref, sc_ref, pos_ref, ps_ref, be_ref,
               ohT_ref, cs_ref):
    T = x_ref.shape[0]
    nt = T // TM
    nc = 2 * nt
    E = NUM_EXPERTS
    NB = be_ref.shape[1]

    for t in range(nt):
        xt = x_ref[pl.ds(t * TM, TM), :]
        logits = jnp.dot(xt, rwt_ref[...],
                         preferred_element_type=jnp.float32) + rb_ref[...]
        lane = jax.lax.broadcasted_iota(jnp.int32, (TM, E), 1)
        m1 = jnp.max(logits, axis=1, keepdims=True)
        idx1 = jnp.min(jnp.where(logits >= m1, lane, E), axis=1, keepdims=True)
        f1 = lane == idx1
        l2 = jnp.where(f1, -jnp.inf, logits)
        m2 = jnp.max(l2, axis=1, keepdims=True)
        idx2 = jnp.min(jnp.where(l2 >= m2, lane, E), axis=1, keepdims=True)
        f2 = lane == idx2
        p1 = 1.0 / (1.0 + jnp.exp(m2 - m1))
        p2 = 1.0 - p1
        sc_ref[pl.ds(t * TM, TM), :] = (
            jnp.where(f1, p1, 0.0) + jnp.where(f2, p2, 0.0))
        f1T = jax.lax.transpose(f1.astype(jnp.float32), (1, 0))
        f2T = jax.lax.transpose(f2.astype(jnp.float32), (1, 0))
        ohT_ref[pl.ds(t * E, E), :] = f1T
        ohT_ref[pl.ds((nt + t) * E, E), :] = f2T
        cs_ref[pl.ds(t, 1), :] = jax.lax.transpose(
            jnp.sum(f1T, axis=1, keepdims=True), (1, 0))
        cs_ref[pl.ds(nt + t, 1), :] = jax.lax.transpose(
            jnp.sum(f2T, axis=1, keepdims=True), (1, 0))
        ps_ref[pl.ds(t, 1), :] = jax.lax.transpose(p1, (1, 0))
        ps_ref[pl.ds(nt + t, 1), :] = jax.lax.transpose(p2, (1, 0))

    cs = cs_ref[...]
    counts = jnp.sum(cs, axis=0, keepdims=True)
    pc = jnp.floor((counts + (TBLK - 1)) / TBLK) * TBLK
    re_ = jax.lax.broadcasted_iota(jnp.int32, (E, E), 0)
    ce = jax.lax.broadcasted_iota(jnp.int32, (E, E), 1)
    u_strict = (re_ < ce).astype(jnp.float32)
    gs = jnp.dot(pc, u_strict, preferred_element_type=jnp.float32)
    rc = jax.lax.broadcasted_iota(jnp.int32, (nc, nc), 0)
    cc = jax.lax.broadcasted_iota(jnp.int32, (nc, nc), 1)
    tri_nc = (rc < cc).astype(jnp.float32)
    base = jnp.dot(jax.lax.transpose(tri_nc, (1, 0)), cs,
                   preferred_element_type=jnp.float32)
    gsT = jax.lax.transpose(gs, (1, 0))
    blkpos = jax.lax.broadcasted_iota(
        jnp.int32, (1, NB), 1).astype(jnp.float32) * float(TBLK)
    be_ref[...] = (jnp.sum((gsT <= blkpos).astype(jnp.int32), axis=0,
                           keepdims=True) - 1)
    rs = jax.lax.broadcasted_iota(jnp.int32, (TM, TM), 0)
    ss = jax.lax.broadcasted_iota(jnp.int32, (TM, TM), 1)
    tri_s = (rs < ss).astype(jnp.float32)
    for c in range(nc):
        ohTc = ohT_ref[pl.ds(c * E, E), :]
        rankT = jnp.dot(ohTc, tri_s, preferred_element_type=jnp.float32)
        baseTc = jax.lax.transpose(base[c:c + 1, :], (1, 0))
        posT = jnp.sum(ohTc * (gsT + baseTc + rankT), axis=0, keepdims=True)
        pos_ref[pl.ds(c, 1), :] = posT.astype(jnp.int32)


def _plan(hs, rwt, rb, nsp):
    T, H = hs.shape
    E = NUM_EXPERTS
    nc = 2 * (T // TM)
    nb = nsp // TBLK
    return pl.pallas_call(
        _plan_body,
        grid=(1,),
        in_specs=[
            pl.BlockSpec((T, H), lambda i: (0, 0)),
            pl.BlockSpec((H, E), lambda i: (0, 0)),
            pl.BlockSpec((1, E), lambda i: (0, 0)),
        ],
        out_specs=[
            pl.BlockSpec((T, E), lambda i: (0, 0)),
            pl.BlockSpec((nc, TM), lambda i: (0, 0)),
            pl.BlockSpec((nc, TM), lambda i: (0, 0)),
            pl.BlockSpec((1, nb), lambda i: (0, 0)),
        ],
        out_shape=[
            jax.ShapeDtypeStruct((T, E), jnp.float32),
            jax.ShapeDtypeStruct((nc, TM), jnp.int32),
            jax.ShapeDtypeStruct((nc, TM), jnp.float32),
            jax.ShapeDtypeStruct((1, nb), jnp.int32),
        ],
        scratch_shapes=[
            pltpu.VMEM((nc * E, TM), jnp.float32),
            pltpu.VMEM((nc, E), jnp.float32),
        ],
    )(hs, rwt, rb)


def _experts_body(be_ref, xs_ref, wgu_ref, bgu_ref, wd_ref, db_ref, ws_ref,
                  y_ref):
    xt = xs_ref[...].astype(jnp.bfloat16)
    wgu = wgu_ref[0].astype(jnp.bfloat16)
    guT = jax.lax.dot_general(
        wgu, xt, (((0,), (1,)), ((), ())),
        preferred_element_type=jnp.float32)
    guT = guT + jax.lax.transpose(bgu_ref[0], (1, 0))
    pair = pltpu.bitcast(guT.astype(jnp.bfloat16), jnp.uint32)
    gate = pltpu.bitcast(pair << 16, jnp.float32)
    up = pltpu.bitcast(pair & jnp.uint32(0xFFFF0000), jnp.float32)
    gate = jnp.minimum(gate, LIMIT)
    up = jnp.clip(up, -LIMIT, LIMIT)
    glu = gate / (1.0 + jnp.exp(-ALPHA * gate))
    act = ((up + 1.0) * glu).astype(jnp.bfloat16)
    wd = wd_ref[0].astype(jnp.bfloat16)
    partial = jax.lax.dot_general(
        act, wd, (((0,), (0,)), ((), ())),
        preferred_element_type=jnp.float32)
    wcol = jax.lax.transpose(ws_ref[0], (1, 0))
    y_ref[...] = (partial + db_ref[0]) * wcol


def _experts(be, xs, wgu, bgu, wd, db, ws3):
    nsp, H = xs.shape
    E, _, I2 = wgu.shape
    I = I2 // 2
    nb = nsp // TBLK
    grid_spec = pltpu.PrefetchScalarGridSpec(
        num_scalar_prefetch=1,
        grid=(nb,),
        in_specs=[
            pl.BlockSpec((TBLK, H), lambda b, be: (b, 0)),
            pl.BlockSpec((1, H, 2 * I), lambda b, be: (be[b], 0, 0)),
            pl.BlockSpec((1, 1, 2 * I), lambda b, be: (be[b], 0, 0)),
            pl.BlockSpec((1, I, H), lambda b, be: (be[b], 0, 0)),
            pl.BlockSpec((1, 1, H), lambda b, be: (be[b], 0, 0)),
            pl.BlockSpec((1, 1, TBLK), lambda b, be: (b, 0, 0)),
        ],
        out_specs=pl.BlockSpec((TBLK, H), lambda b, be: (b, 0)),
    )
    return pl.pallas_call(
        _experts_body,
        grid_spec=grid_spec,
        out_shape=jax.ShapeDtypeStruct((nsp, H), jnp.float32),
        compiler_params=pltpu.CompilerParams(
            dimension_semantics=("arbitrary",),
        ),
    )(be, xs, wgu, bgu, wd, db, ws3)


def _dispatch(pos2, ps2, hs, nsp):
    T, H = hs.shape
    nc, TMc = pos2.shape
    per_w = nc * TMc // 32
    half = per_w // 2
    mesh = plsc.VectorSubcoreMesh(core_axis_name="c", subcore_axis_name="s")

    @functools.partial(
        pl.kernel, mesh=mesh,
        out_type=[
            jax.ShapeDtypeStruct((nsp, H), jnp.float32),
            jax.ShapeDtypeStruct((nsp,), jnp.float32),
        ],
        scratch_types=[
            pltpu.VMEM((per_w,), jnp.int32),
            pltpu.VMEM((half,), jnp.int32),
            pltpu.VMEM((half,), jnp.int32),
            pltpu.VMEM((half, H), jnp.float32),
            pltpu.VMEM((half, H), jnp.float32),
            pltpu.VMEM((per_w,), jnp.float32),
            pltpu.SemaphoreType.DMA,
            pltpu.SemaphoreType.DMA,
            pltpu.SemaphoreType.DMA,
        ],
    )
    def k(pos_hbm, ps_hbm, x_hbm, xs_hbm, ws_hbm,
          idx_v, idx_a, idx_b, rows_a, rows_b, p_v, sem_a, sem_b, sem_s):
        wid = lax.axis_index("s") * 2 + lax.axis_index("c")
        s0 = wid * per_w
        row = wid // 2
        col = lax.rem(wid, 2) * per_w
        t0 = lax.rem(s0, T)
        pltpu.sync_copy(pos_hbm.at[row, pl.ds(col, per_w)], idx_v)
        pltpu.sync_copy(pos_hbm.at[row, pl.ds(col, half)], idx_a)
        pltpu.sync_copy(pos_hbm.at[row, pl.ds(col + half, half)], idx_b)
        pltpu.sync_copy(ps_hbm.at[row, pl.ds(col, per_w)], p_v)
        cp_a = pltpu.async_copy(x_hbm.at[pl.ds(t0, half)], rows_a, sem_a)
        cp_b = pltpu.async_copy(x_hbm.at[pl.ds(t0 + half, half)], rows_b,
                                sem_b)
        pltpu.async_copy(p_v, ws_hbm.at[idx_v], sem_s).wait()
        cp_a.wait()
        sc_a = pltpu.async_copy(rows_a, xs_hbm.at[idx_a], sem_a)
        cp_b.wait()
        pltpu.async_copy(rows_b, xs_hbm.at[idx_b], sem_b).wait()
        sc_a.wait()

    return k(pos2, ps2, hs)


def _combine(pos2, y, T):
    nsp, H = y.shape
    nc, TMc = pos2.shape
    per_w = T // 32
    rows_per = TMc // per_w
    mesh = plsc.VectorSubcoreMesh(core_axis_name="c", subcore_axis_name="s")

    @functools.partial(
        pl.kernel, mesh=mesh,
        out_type=jax.ShapeDtypeStruct((T, H), jnp.float32),
        scratch_types=[
            pltpu.VMEM((per_w,), jnp.int32),
            pltpu.VMEM((per_w,), jnp.int32),
            pltpu.VMEM((per_w, H), jnp.float32),
            pltpu.VMEM((per_w, H), jnp.float32),
            pltpu.SemaphoreType.DMA,
        ],
    )
    def k(pos_hbm, y_hbm, out_hbm, i1_v, i2_v, r1_v, r2_v, sem):
        wid = lax.axis_index("s") * 2 + lax.axis_index("c")
        t0 = wid * per_w
        row = wid // rows_per
        col = lax.rem(wid, rows_per) * per_w
        pltpu.sync_copy(pos_hbm.at[row, pl.ds(col, per_w)], i1_v)
        pltpu.sync_copy(pos_hbm.at[nc // 2 + row, pl.ds(col, per_w)], i2_v)
        pltpu.async_copy(y_hbm.at[i1_v], r1_v, sem).wait()
        pltpu.async_copy(y_hbm.at[i2_v], r2_v, sem).wait()

        def row(i, _):
            for j in range(H // 16):
                cols = pl.ds(j * 16, 16)
                r1_v[i, cols] = r1_v[i, cols] + r2_v[i, cols]
            return 0

        lax.fori_loop(0, per_w, row, 0)
        pltpu.sync_copy(r1_v, out_hbm.at[pl.ds(t0, per_w)])

    return k(pos2, y)


@functools.partial(jax.jit, static_argnames=())
def _moe(hs, rwt, rb, wgu, bgu, wd, db):
    T, H = hs.shape
    I = wd.shape[1]
    nsp = 2 * T + NUM_EXPERTS * TBLK - NUM_EXPERTS * 1
    nsp = ((nsp + TBLK - 1) // TBLK) * TBLK
    scores, pos, ps, be = _plan(hs, rwt, rb, nsp)
    xs, ws = _dispatch(pos, ps, hs, nsp)
    y = _experts(be.reshape(nsp // TBLK), xs, wgu, bgu, wd, db,
                 ws.reshape(nsp // TBLK, 1, TBLK))
    out = _combine(pos, y, T)
    return out, scores


def kernel(hidden_states, router_weight, router_bias, gate_up_proj,
           gate_up_bias, down_proj, down_bias):
    B, S, H = hidden_states.shape
    E = NUM_EXPERTS
    I = down_proj.shape[1]
    hs = hidden_states.reshape(B * S, H)
    rwt = router_weight.T
    rb = router_bias.reshape(1, E)
    bgu = gate_up_bias.reshape(E, 1, 2 * I)
    db = down_bias.reshape(E, 1, H)
    out, scores = _moe(hs, rwt, rb, gate_up_proj, bgu, down_proj, db)
    return out.reshape(B, S, H), scores
